# Initial kernel scaffold; baseline (speedup 1.0000x reference)
import jax
import jax.numpy as jnp
from jax.experimental import pallas as pl


def kernel(x, edge_index, eps, W1, b1, gamma, beta, W2, b2):
    raise NotImplementedError("write your pallas kernel here")


# speedup vs baseline: 5.2698x; 5.2698x over previous
import functools

import jax
import jax.numpy as jnp
from jax import lax
from jax.experimental import pallas as pl
from jax.experimental.pallas import tpu as pltpu
from jax.experimental.pallas import tpu_sc as plsc

N_NODES = 10000
D_FEAT = 128
N_EDGES = 320000
N_CLASSES = 40

NC = 2
NS = 16
NW = NC * NS

EDGES_PER_WORKER = N_EDGES // NW
CHUNK = 80
NCHUNKS = EDGES_PER_WORKER // CHUNK
ROWS_PER_SUBCORE = 624
TAIL_ROWS = N_NODES - NS * ROWS_PER_SUBCORE
TAIL_BASE = NS * ROWS_PER_SUBCORE


def _sc_partial_agg(x, src, dst, zeros):
  mesh = plsc.VectorSubcoreMesh(core_axis_name="c", subcore_axis_name="s")

  @functools.partial(
      pl.kernel,
      out_type=jax.ShapeDtypeStruct((NC, N_NODES, D_FEAT), jnp.float32),
      mesh=mesh,
      scratch_types=[
          pltpu.VMEM((CHUNK,), jnp.int32),
          pltpu.VMEM((CHUNK,), jnp.int32),
          pltpu.VMEM((CHUNK, D_FEAT), jnp.float32),
          pltpu.VMEM_SHARED((N_NODES, D_FEAT), jnp.float32),
          pltpu.SemaphoreType.DMA,
      ],
  )
  def k(x_hbm, src_hbm, dst_hbm, zeros_hbm, out_hbm, sidx, didx, rows, accum,
        sem):
    cid = lax.axis_index("c")
    sid = lax.axis_index("s")
    wid = sid * NC + cid
    ebase = wid * EDGES_PER_WORKER
    rbase = sid * ROWS_PER_SUBCORE

    pltpu.sync_copy(zeros_hbm.at[pl.ds(rbase, ROWS_PER_SUBCORE)],
                    accum.at[pl.ds(rbase, ROWS_PER_SUBCORE)])

    @pl.when(sid == NS - 1)
    def _():
      pltpu.sync_copy(zeros_hbm.at[pl.ds(TAIL_BASE, TAIL_ROWS)],
                      accum.at[pl.ds(TAIL_BASE, TAIL_ROWS)])

    plsc.subcore_barrier()

    def body(i):
      off = ebase + i * CHUNK
      pltpu.sync_copy(src_hbm.at[pl.ds(off, CHUNK)], sidx)
      pltpu.async_copy(x_hbm.at[sidx], rows, sem).wait()
      pltpu.sync_copy(dst_hbm.at[pl.ds(off, CHUNK)], didx)
      pltpu.sync_copy(rows, accum.at[didx], add=True)

    pl.loop(0, NCHUNKS)(body)
    plsc.subcore_barrier()

    pltpu.sync_copy(accum.at[pl.ds(rbase, ROWS_PER_SUBCORE)],
                    out_hbm.at[cid, pl.ds(rbase, ROWS_PER_SUBCORE)])

    @pl.when(sid == NS - 1)
    def _():
      pltpu.sync_copy(accum.at[pl.ds(TAIL_BASE, TAIL_ROWS)],
                      out_hbm.at[cid, pl.ds(TAIL_BASE, TAIL_ROWS)])

  return k(x, src, dst, zeros)


def _tc_mlp_body(x_ref, a0_ref, a1_ref, eps_ref, w1t_ref, b1_ref, gamma_ref,
                 beta_ref, w2t_ref, b2_ref, out_ref):
  h = (1.0 + eps_ref[0, 0]) * x_ref[...] + a0_ref[...] + a1_ref[...]
  h = jnp.dot(h, w1t_ref[...], preferred_element_type=jnp.float32)
  h = jnp.maximum(h + b1_ref[...], 0.0)
  mean = jnp.mean(h, axis=0, keepdims=True)
  var = jnp.mean(jnp.square(h - mean), axis=0, keepdims=True)
  h = (h - mean) * lax.rsqrt(var + 1e-5) * gamma_ref[...] + beta_ref[...]
  o = jnp.dot(h, w2t_ref[...], preferred_element_type=jnp.float32)
  o = o + b2_ref[...]
  m = jnp.max(o, axis=-1, keepdims=True)
  lse = m + jnp.log(jnp.sum(jnp.exp(o - m), axis=-1, keepdims=True))
  out_ref[...] = o - lse


def _tc_mlp(x, a0, a1, eps, w1t, b1, gamma, beta, w2t, b2):
  return pl.pallas_call(
      _tc_mlp_body,
      out_shape=jax.ShapeDtypeStruct((N_NODES, N_CLASSES), jnp.float32),
      in_specs=[
          pl.BlockSpec(memory_space=pltpu.VMEM),
          pl.BlockSpec(memory_space=pltpu.VMEM),
          pl.BlockSpec(memory_space=pltpu.VMEM),
          pl.BlockSpec(memory_space=pltpu.SMEM),
          pl.BlockSpec(memory_space=pltpu.VMEM),
          pl.BlockSpec(memory_space=pltpu.VMEM),
          pl.BlockSpec(memory_space=pltpu.VMEM),
          pl.BlockSpec(memory_space=pltpu.VMEM),
          pl.BlockSpec(memory_space=pltpu.VMEM),
          pl.BlockSpec(memory_space=pltpu.VMEM),
      ],
      out_specs=pl.BlockSpec(memory_space=pltpu.VMEM),
  )(x, a0, a1, eps, w1t, b1, gamma, beta, w2t, b2)


def kernel(x, edge_index, eps, W1, b1, gamma, beta, W2, b2):
  src = edge_index[0].astype(jnp.int32)
  dst = edge_index[1].astype(jnp.int32)
  zeros = jnp.zeros((N_NODES, D_FEAT), jnp.float32)
  agg = _sc_partial_agg(x, src, dst, zeros)
  eps2d = jnp.reshape(eps.astype(jnp.float32), (1, 1))
  out = _tc_mlp(x, agg[0], agg[1], eps2d, W1.T, jnp.reshape(b1, (1, -1)),
                jnp.reshape(gamma, (1, -1)), jnp.reshape(beta, (1, -1)),
                W2.T, jnp.reshape(b2, (1, -1)))
  return out


# speedup vs baseline: 6.1568x; 1.1683x over previous
import functools

import jax
import jax.numpy as jnp
from jax import lax
from jax.experimental import pallas as pl
from jax.experimental.pallas import tpu as pltpu
from jax.experimental.pallas import tpu_sc as plsc

N_NODES = 10000
D_FEAT = 128
N_EDGES = 320000
N_CLASSES = 40

NC = 2
NS = 16
NW = NC * NS

GCH = 64
SROW = 128
SROWS = 79
EPW = SROWS * SROW
NCH = EPW // GCH
PAD_EDGES = NW * EPW - N_EDGES
ROWS_PER_SUBCORE = 624
TAIL_ROWS = N_NODES - NS * ROWS_PER_SUBCORE
TAIL_BASE = NS * ROWS_PER_SUBCORE


def _sc_partial_agg(xz, src, dst, zeros):
  mesh = plsc.VectorSubcoreMesh(core_axis_name="c", subcore_axis_name="s")

  @functools.partial(
      pl.kernel,
      out_type=jax.ShapeDtypeStruct((NC, N_NODES, D_FEAT), jnp.float32),
      mesh=mesh,
      scratch_types=[
          pltpu.VMEM((SROWS, SROW), jnp.int32),
          pltpu.VMEM((NCH, GCH), jnp.int32),
          pltpu.VMEM((GCH, D_FEAT), jnp.float32),
          pltpu.VMEM((GCH, D_FEAT), jnp.float32),
          pltpu.VMEM_SHARED((N_NODES, D_FEAT), jnp.float32),
          pltpu.SemaphoreType.DMA,
          pltpu.SemaphoreType.DMA,
      ],
  )
  def k(x_hbm, src_hbm, dst_hbm, zeros_hbm, out_hbm, sidx, didx, rows_a,
        rows_b, accum, sem_a, sem_b):
    cid = lax.axis_index("c")
    sid = lax.axis_index("s")
    wid = sid * NC + cid
    rbase = sid * ROWS_PER_SUBCORE

    pltpu.sync_copy(src_hbm.at[wid], sidx)
    pltpu.sync_copy(dst_hbm.at[wid], didx)
    pltpu.sync_copy(zeros_hbm.at[pl.ds(rbase, ROWS_PER_SUBCORE)],
                    accum.at[pl.ds(rbase, ROWS_PER_SUBCORE)])

    @pl.when(sid == NS - 1)
    def _():
      pltpu.sync_copy(zeros_hbm.at[pl.ds(TAIL_BASE, TAIL_ROWS)],
                      accum.at[pl.ds(TAIL_BASE, TAIL_ROWS)])

    plsc.subcore_barrier()

    def gather(r, c, rows, sem):
      pltpu.async_copy(x_hbm.at[sidx.at[r, pl.ds(c, GCH)]], rows, sem)

    def drain(r, c, rows, sem):
      pltpu.make_async_copy(x_hbm.at[sidx.at[r, pl.ds(c, GCH)]], rows,
                            sem).wait()

    def scatter(i, rows):
      pltpu.sync_copy(rows, accum.at[didx.at[i]], add=True)

    gather(0, 0, rows_a, sem_a)

    def body(j):
      gather(j, GCH, rows_b, sem_b)
      drain(j, 0, rows_a, sem_a)
      scatter(2 * j, rows_a)
      gather(j + 1, 0, rows_a, sem_a)
      drain(j, GCH, rows_b, sem_b)
      scatter(2 * j + 1, rows_b)

    pl.loop(0, NCH // 2 - 1)(body)
    gather(NCH // 2 - 1, GCH, rows_b, sem_b)
    drain(NCH // 2 - 1, 0, rows_a, sem_a)
    scatter(NCH - 2, rows_a)
    drain(NCH // 2 - 1, GCH, rows_b, sem_b)
    scatter(NCH - 1, rows_b)
    plsc.subcore_barrier()

    pltpu.sync_copy(accum.at[pl.ds(rbase, ROWS_PER_SUBCORE)],
                    out_hbm.at[cid, pl.ds(rbase, ROWS_PER_SUBCORE)])

    @pl.when(sid == NS - 1)
    def _():
      pltpu.sync_copy(accum.at[pl.ds(TAIL_BASE, TAIL_ROWS)],
                      out_hbm.at[cid, pl.ds(TAIL_BASE, TAIL_ROWS)])

  return k(xz, src, dst, zeros)


def _tc_mlp_body(x_ref, a0_ref, a1_ref, eps_ref, w1t_ref, b1_ref, gamma_ref,
                 beta_ref, w2t_ref, b2_ref, out_ref):
  h = (1.0 + eps_ref[0, 0]) * x_ref[...] + a0_ref[...] + a1_ref[...]
  h = jnp.dot(h, w1t_ref[...], preferred_element_type=jnp.float32)
  h = jnp.maximum(h + b1_ref[...], 0.0)
  mean = jnp.mean(h, axis=0, keepdims=True)
  var = jnp.mean(jnp.square(h - mean), axis=0, keepdims=True)
  h = (h - mean) * lax.rsqrt(var + 1e-5) * gamma_ref[...] + beta_ref[...]
  o = jnp.dot(h, w2t_ref[...], preferred_element_type=jnp.float32)
  o = o + b2_ref[...]
  m = jnp.max(o, axis=-1, keepdims=True)
  lse = m + jnp.log(jnp.sum(jnp.exp(o - m), axis=-1, keepdims=True))
  out_ref[...] = o - lse


def _tc_mlp(x, a0, a1, eps, w1t, b1, gamma, beta, w2t, b2):
  return pl.pallas_call(
      _tc_mlp_body,
      out_shape=jax.ShapeDtypeStruct((N_NODES, N_CLASSES), jnp.float32),
      in_specs=[
          pl.BlockSpec(memory_space=pltpu.VMEM),
          pl.BlockSpec(memory_space=pltpu.VMEM),
          pl.BlockSpec(memory_space=pltpu.VMEM),
          pl.BlockSpec(memory_space=pltpu.SMEM),
          pl.BlockSpec(memory_space=pltpu.VMEM),
          pl.BlockSpec(memory_space=pltpu.VMEM),
          pl.BlockSpec(memory_space=pltpu.VMEM),
          pl.BlockSpec(memory_space=pltpu.VMEM),
          pl.BlockSpec(memory_space=pltpu.VMEM),
          pl.BlockSpec(memory_space=pltpu.VMEM),
      ],
      out_specs=pl.BlockSpec(memory_space=pltpu.VMEM),
  )(x, a0, a1, eps, w1t, b1, gamma, beta, w2t, b2)


def kernel(x, edge_index, eps, W1, b1, gamma, beta, W2, b2):
  src = jnp.concatenate([
      edge_index[0].astype(jnp.int32),
      jnp.full((PAD_EDGES,), N_NODES, jnp.int32),
  ]).reshape(NW, SROWS, SROW)
  dst = jnp.concatenate([
      edge_index[1].astype(jnp.int32),
      jnp.zeros((PAD_EDGES,), jnp.int32),
  ]).reshape(NW, NCH, GCH)
  xz = jnp.concatenate([x, jnp.zeros((1, D_FEAT), jnp.float32)], axis=0)
  zeros = jnp.zeros((N_NODES, D_FEAT), jnp.float32)
  agg = _sc_partial_agg(xz, src, dst, zeros)
  eps2d = jnp.reshape(eps.astype(jnp.float32), (1, 1))
  out = _tc_mlp(x, agg[0], agg[1], eps2d, W1.T, jnp.reshape(b1, (1, -1)),
                jnp.reshape(gamma, (1, -1)), jnp.reshape(beta, (1, -1)),
                W2.T, jnp.reshape(b2, (1, -1)))
  return out
